# baseline (speedup 1.0000x reference)
import functools

import jax
import jax.numpy as jnp
from jax.experimental import pallas as pl
from jax.experimental.pallas import tpu as pltpu

_dot = functools.partial(jnp.dot, preferred_element_type=jnp.float32)
_dot_hi = functools.partial(jnp.dot, preferred_element_type=jnp.float32,
                            precision=jax.lax.Precision.HIGHEST)

B, T, A, F_IN = 32, 128, 22, 8
EMB, HID, LH = 32, 32, 128
G4 = 4 * LH
AF = A * F_IN
AE = A * EMB


def _fused_kernel(feat_ref, w_big_ref, b_big_ref, sw1_ref, b_g1_ref,
                  w_g2_ref, b_g2_ref, wih_ft_ref, whh_ft_ref, bf_ref,
                  wih_rt_ref, br_ref, w_cls_ref, b_cls_ref,
                  out_ref, u_ref):
    e = jax.nn.relu(
        _dot_hi(feat_ref[...], w_big_ref[...]) + b_big_ref[...])
    g1 = jax.nn.relu(
        _dot_hi(e, sw1_ref[...])
        + b_g1_ref[...])
    g2 = jax.nn.relu(
        _dot_hi(g1, w_g2_ref[...])
        + b_g2_ref[...])

    u_ref[...] = _dot(g2, wih_ft_ref[...]) + bf_ref[...]

    whh_ft = whh_ft_ref[...]

    def step(t, carry):
        h, c = carry
        g = u_ref[pl.ds(t * B, B), :] + _dot(h, whh_ft)
        i = jax.nn.sigmoid(g[:, 0 * LH:1 * LH])
        f = jax.nn.sigmoid(g[:, 1 * LH:2 * LH])
        gg = jnp.tanh(g[:, 2 * LH:3 * LH])
        o = jax.nn.sigmoid(g[:, 3 * LH:4 * LH])
        c = f * c + i * gg
        h = o * jnp.tanh(c)
        return (h, c)

    h0 = jnp.zeros((B, LH), jnp.float32)
    c0 = jnp.zeros((B, LH), jnp.float32)
    h_f, _ = jax.lax.fori_loop(0, T, step, (h0, c0), unroll=8)

    x_last = g2[(T - 1) * B:, :]
    gr = _dot(x_last, wih_rt_ref[...]) + br_ref[...]
    cr = jax.nn.sigmoid(gr[:, 0 * LH:1 * LH]) * jnp.tanh(gr[:, 2 * LH:3 * LH])
    h_r = jax.nn.sigmoid(gr[:, 3 * LH:4 * LH]) * jnp.tanh(cr)

    last = jnp.concatenate([h_f, h_r], axis=1)
    out_ref[...] = _dot(last, w_cls_ref[...]) + b_cls_ref[...]


def kernel(features, pressing_intensity, agent_order, W_embed, b_embed,
           W_g1, b_g1, W_g2, b_g2, Wih_f, Whh_f, bih_f, bhh_f,
           Wih_r, Whh_r, bih_r, bhh_r, W_cls, b_cls):
    feat2 = jnp.transpose(features, (1, 0, 2, 3)).reshape(T * B, AF)
    w_big = jax.scipy.linalg.block_diag(*([W_embed] * A))
    b_big = jnp.tile(b_embed, A).reshape(1, AE)
    sw1 = jnp.tile(W_g1, (A, 1)) * (1.0 / A)
    bf = (bih_f + bhh_f).reshape(1, G4)
    br = (bih_r + bhh_r).reshape(1, G4)
    return pl.pallas_call(
        _fused_kernel,
        out_shape=jax.ShapeDtypeStruct((B, 1), jnp.float32),
        scratch_shapes=[pltpu.VMEM((T * B, G4), jnp.float32)],
    )(feat2, w_big, b_big, sw1, b_g1.reshape(1, HID),
      W_g2, b_g2.reshape(1, HID),
      Wih_f.T, Whh_f.T, bf, Wih_r.T, br, W_cls, b_cls.reshape(1, 1))


# speedup vs baseline: 1.0730x; 1.0730x over previous
import functools

import jax
import jax.numpy as jnp
from jax.experimental import pallas as pl
from jax.experimental.pallas import tpu as pltpu

_dot = functools.partial(jnp.dot, preferred_element_type=jnp.float32)
_dot_hi = functools.partial(jnp.dot, preferred_element_type=jnp.float32,
                            precision=jax.lax.Precision.HIGH)

B, T, A, F_IN = 32, 128, 22, 8
EMB, HID, LH = 32, 32, 128
G4 = 4 * LH
AF = A * F_IN
AE = A * EMB


def _fused_kernel(feat_ref, w_emb_ref, b_emb_ref, w_g1_ref, b_g1_ref,
                  w_g2_ref, b_g2_ref, wih_ft_ref, whh_ft_ref, bf_ref,
                  wih_rt_ref, br_ref, w_cls_ref, b_cls_ref,
                  out_ref, u_ref):
    w_emb = w_emb_ref[...]
    b_emb = b_emb_ref[...]
    acc = jax.nn.relu(_dot(feat_ref[0], w_emb) + b_emb)
    for a in range(1, A):
        acc = acc + jax.nn.relu(_dot(feat_ref[a], w_emb) + b_emb)
    m = acc * (1.0 / A)
    g1 = jax.nn.relu(
        _dot(m, w_g1_ref[...])
        + b_g1_ref[...])
    g2 = jax.nn.relu(
        _dot(g1, w_g2_ref[...])
        + b_g2_ref[...])

    u_ref[...] = _dot(g2, wih_ft_ref[...]) + bf_ref[...]

    whh_ft = whh_ft_ref[...]

    def step(t, carry):
        h, c = carry
        g = u_ref[pl.ds(t * B, B), :] + _dot(h, whh_ft)
        i = jax.nn.sigmoid(g[:, 0 * LH:1 * LH])
        f = jax.nn.sigmoid(g[:, 1 * LH:2 * LH])
        gg = jnp.tanh(g[:, 2 * LH:3 * LH])
        o = jax.nn.sigmoid(g[:, 3 * LH:4 * LH])
        c = f * c + i * gg
        h = o * jnp.tanh(c)
        return (h, c)

    h0 = jnp.zeros((B, LH), jnp.float32)
    c0 = jnp.zeros((B, LH), jnp.float32)
    h_f, _ = jax.lax.fori_loop(0, T, step, (h0, c0), unroll=8)

    x_last = g2[(T - 1) * B:, :]
    gr = _dot(x_last, wih_rt_ref[...]) + br_ref[...]
    cr = jax.nn.sigmoid(gr[:, 0 * LH:1 * LH]) * jnp.tanh(gr[:, 2 * LH:3 * LH])
    h_r = jax.nn.sigmoid(gr[:, 3 * LH:4 * LH]) * jnp.tanh(cr)

    last = jnp.concatenate([h_f, h_r], axis=1)
    out_ref[...] = _dot(last, w_cls_ref[...]) + b_cls_ref[...]


def kernel(features, pressing_intensity, agent_order, W_embed, b_embed,
           W_g1, b_g1, W_g2, b_g2, Wih_f, Whh_f, bih_f, bhh_f,
           Wih_r, Whh_r, bih_r, bhh_r, W_cls, b_cls):
    feat = jnp.transpose(features, (2, 1, 0, 3)).reshape(A, T * B, F_IN)
    bf = (bih_f + bhh_f).reshape(1, G4)
    br = (bih_r + bhh_r).reshape(1, G4)
    return pl.pallas_call(
        _fused_kernel,
        out_shape=jax.ShapeDtypeStruct((B, 1), jnp.float32),
        scratch_shapes=[pltpu.VMEM((T * B, G4), jnp.float32)],
    )(feat, W_embed, b_embed.reshape(1, EMB), W_g1, b_g1.reshape(1, HID),
      W_g2, b_g2.reshape(1, HID),
      Wih_f.T, Whh_f.T, bf, Wih_r.T, br, W_cls, b_cls.reshape(1, 1))


# speedup vs baseline: 1.4476x; 1.3491x over previous
import functools

import jax
import jax.numpy as jnp
from jax.experimental import pallas as pl
from jax.experimental.pallas import tpu as pltpu

_dot = functools.partial(jnp.dot, preferred_element_type=jnp.float32)

B, T, A, F_IN = 32, 128, 22, 8
EMB, HID, LH = 32, 32, 128
G4 = 4 * LH
AF = A * F_IN
AP = 24
AEP = AP * EMB


def _fused_kernel(feat_ref, w_big_ref, b_big_ref, w_g1_ref, b_g1_ref,
                  w_g2_ref, b_g2_ref, wih_ft_ref, whh_ft_ref, bf_ref,
                  wih_rt_ref, br_ref, w_cls_ref, b_cls_ref,
                  out_ref, u_ref):
    e = jax.nn.relu(
        _dot(feat_ref[...], w_big_ref[...]) + b_big_ref[...])
    w = e[:, 0:128]
    for k in range(1, AEP // 128):
        w = w + e[:, 128 * k:128 * (k + 1)]
    m = (w[:, 0:32] + w[:, 32:64] + w[:, 64:96] + w[:, 96:128]) * (1.0 / A)
    g1 = jax.nn.relu(
        _dot(m, w_g1_ref[...]) + b_g1_ref[...])
    g2 = jax.nn.relu(
        _dot(g1, w_g2_ref[...]) + b_g2_ref[...])

    u_ref[...] = _dot(g2, wih_ft_ref[...]) + bf_ref[...]

    whh_ft = whh_ft_ref[...]

    def step(t, carry):
        h, c = carry
        g = u_ref[pl.ds(t * B, B), :] + _dot(h, whh_ft)
        i = jax.nn.sigmoid(g[:, 0 * LH:1 * LH])
        f = jax.nn.sigmoid(g[:, 1 * LH:2 * LH])
        gg = jnp.tanh(g[:, 2 * LH:3 * LH])
        o = jax.nn.sigmoid(g[:, 3 * LH:4 * LH])
        c = f * c + i * gg
        h = o * jnp.tanh(c)
        return (h, c)

    h0 = jnp.zeros((B, LH), jnp.float32)
    c0 = jnp.zeros((B, LH), jnp.float32)
    h_f, _ = jax.lax.fori_loop(0, T, step, (h0, c0), unroll=8)

    x_last = g2[(T - 1) * B:, :]
    gr = _dot(x_last, wih_rt_ref[...]) + br_ref[...]
    cr = jax.nn.sigmoid(gr[:, 0 * LH:1 * LH]) * jnp.tanh(gr[:, 2 * LH:3 * LH])
    h_r = jax.nn.sigmoid(gr[:, 3 * LH:4 * LH]) * jnp.tanh(cr)

    last = jnp.concatenate([h_f, h_r], axis=1)
    out_ref[...] = _dot(last, w_cls_ref[...]) + b_cls_ref[...]


def kernel(features, pressing_intensity, agent_order, W_embed, b_embed,
           W_g1, b_g1, W_g2, b_g2, Wih_f, Whh_f, bih_f, bhh_f,
           Wih_r, Whh_r, bih_r, bhh_r, W_cls, b_cls):
    feat2 = jnp.transpose(features, (1, 0, 2, 3)).reshape(T * B, AF)
    w_big = jnp.zeros((AF, AEP), jnp.float32)
    w_big = w_big.at[:, :A * EMB].set(
        jax.scipy.linalg.block_diag(*([W_embed] * A)))
    b_big = jnp.zeros((1, AEP), jnp.float32)
    b_big = b_big.at[0, :A * EMB].set(jnp.tile(b_embed, A))
    bf = (bih_f + bhh_f).reshape(1, G4)
    br = (bih_r + bhh_r).reshape(1, G4)
    return pl.pallas_call(
        _fused_kernel,
        out_shape=jax.ShapeDtypeStruct((B, 1), jnp.float32),
        scratch_shapes=[pltpu.VMEM((T * B, G4), jnp.float32)],
    )(feat2, w_big, b_big, W_g1, b_g1.reshape(1, HID),
      W_g2, b_g2.reshape(1, HID),
      Wih_f.T, Whh_f.T, bf, Wih_r.T, br, W_cls, b_cls.reshape(1, 1))
